# Initial kernel scaffold; baseline (speedup 1.0000x reference)
import jax
import jax.numpy as jnp
from jax.experimental import pallas as pl


def kernel(x, edge_index, edge_type, W_rel1, W_root1, b1, W_rel2, W_root2, b2):
    raise NotImplementedError("write your pallas kernel here")


# speedup vs baseline: 11.6794x; 11.6794x over previous
import functools

import jax
import jax.numpy as jnp
from jax import lax
from jax.experimental import pallas as pl
from jax.experimental.pallas import tpu as pltpu
from jax.experimental.pallas import tpu_sc as plsc

N = 10000
E = 320000
D = 128
R = 8

NC = 2
NS = 16
NW = NC * NS
EPW = E // NW
EPS = E // NS
CH = 80
NCH_W = EPW // CH
NCH_S = EPS // CH
RN = R * N
RN_PAD = 81920
CNT_SLICE = RN_PAD // NS
RPT = N // NS

_mesh = plsc.VectorSubcoreMesh(core_axis_name="c", subcore_axis_name="s")


BN = 400
NB = N // BN


def _mm_body(x_ref, w_ref, b_ref, out_ref):
    r = pl.program_id(0)
    acc = jnp.dot(x_ref[...], w_ref[0], preferred_element_type=jnp.float32)
    bias = b_ref[0] * (r == R).astype(jnp.float32)
    out_ref[0] = acc + bias[None, :]


def _mm_relu_body(a_ref, w_ref, b_ref, out_ref):
    r = pl.program_id(0)
    xb = jnp.maximum(a_ref[0] + a_ref[1], 0.0)
    acc = jnp.dot(xb, w_ref[0], preferred_element_type=jnp.float32)
    bias = b_ref[0] * (r == R).astype(jnp.float32)
    out_ref[0] = acc + bias[None, :]


def _matmul_all(x, w_all, b):
    return pl.pallas_call(
        _mm_body,
        grid=(R + 1, NB),
        in_specs=[
            pl.BlockSpec((BN, D), lambda r, i: (i, 0)),
            pl.BlockSpec((1, D, D), lambda r, i: (r, 0, 0)),
            pl.BlockSpec((1, D), lambda r, i: (0, 0)),
        ],
        out_specs=pl.BlockSpec((1, BN, D), lambda r, i: (r, i, 0)),
        out_shape=jax.ShapeDtypeStruct((R + 1, N, D), jnp.float32),
    )(x, w_all, b[None])


def _matmul_all_relu(acc, w_all, b):
    return pl.pallas_call(
        _mm_relu_body,
        grid=(R + 1, NB),
        in_specs=[
            pl.BlockSpec((NC, BN, D), lambda r, i: (0, i, 0)),
            pl.BlockSpec((1, D, D), lambda r, i: (r, 0, 0)),
            pl.BlockSpec((1, D), lambda r, i: (0, 0)),
        ],
        out_specs=pl.BlockSpec((1, BN, D), lambda r, i: (r, i, 0)),
        out_shape=jax.ShapeDtypeStruct((R + 1, N, D), jnp.float32),
    )(acc, w_all, b[None])


def _idx_body(src_ref, dst_ref, et_ref, fs_ref, fd_ref):
    base = et_ref[...] * N
    fs_ref[...] = base + src_ref[...]
    fd_ref[...] = base + dst_ref[...]


def _idx_prep(src2, dst2, et2):
    rows = E // D
    return pl.pallas_call(
        _idx_body,
        out_shape=[jax.ShapeDtypeStruct((rows, D), jnp.int32)] * 2,
    )(src2, dst2, et2)


def _sum_body(a_ref, out_ref):
    out_ref[...] = a_ref[0] + a_ref[1]


def _final_sum(acc):
    return pl.pallas_call(
        _sum_body,
        grid=(NB,),
        in_specs=[pl.BlockSpec((NC, BN, D), lambda i: (0, i, 0))],
        out_specs=pl.BlockSpec((BN, D), lambda i: (i, 0)),
        out_shape=jax.ShapeDtypeStruct((N, D), jnp.float32),
    )(acc)


@functools.partial(
    pl.kernel,
    out_type=jax.ShapeDtypeStruct((E,), jnp.float32),
    mesh=_mesh,
    compiler_params=pltpu.CompilerParams(
        needs_layout_passes=False, use_tc_tiling_on_sc=False),
    scratch_types=[
        pltpu.VMEM_SHARED((RN_PAD,), jnp.float32),
        pltpu.VMEM((CNT_SLICE,), jnp.float32),
        pltpu.VMEM((CH,), jnp.float32),
        pltpu.VMEM((CH,), jnp.int32),
        pltpu.VMEM((CH,), jnp.float32),
        pltpu.VMEM((RN,), jnp.float32),
    ],
)
def _weights_kernel(fdst_hbm, w_hbm, cnt_sh, cbuf, ones_v, idx_v, w_v, inv_v):
    c = lax.axis_index("c")
    s = lax.axis_index("s")
    wid = s * NC + c

    def z16(i, _):
        cbuf[pl.ds(i * 16, 16)] = jnp.zeros((16,), jnp.float32)
        return 0
    lax.fori_loop(0, CNT_SLICE // 16, z16, 0)
    pltpu.sync_copy(cbuf, cnt_sh.at[pl.ds(s * CNT_SLICE, CNT_SLICE)])

    def o16(i, _):
        ones_v[pl.ds(i * 16, 16)] = jnp.ones((16,), jnp.float32)
        return 0
    lax.fori_loop(0, CH // 16, o16, 0)
    plsc.subcore_barrier()

    def cnt_step(g, _):
        base = s * EPS + g * CH
        pltpu.sync_copy(fdst_hbm.at[pl.ds(base, CH)], idx_v)
        pltpu.sync_copy(ones_v, cnt_sh.at[idx_v], add=True)
        return 0
    lax.fori_loop(0, NCH_S, cnt_step, 0)
    plsc.subcore_barrier()

    pltpu.sync_copy(cnt_sh.at[pl.ds(s * CNT_SLICE, CNT_SLICE)], cbuf)

    def inv16(i, _):
        v = cbuf[pl.ds(i * 16, 16)]
        cbuf[pl.ds(i * 16, 16)] = 1.0 / jnp.maximum(v, 1.0)
        return 0
    lax.fori_loop(0, CNT_SLICE // 16, inv16, 0)
    pltpu.sync_copy(cbuf, cnt_sh.at[pl.ds(s * CNT_SLICE, CNT_SLICE)])
    plsc.subcore_barrier()

    pltpu.sync_copy(cnt_sh.at[pl.ds(0, RN)], inv_v)

    def w_step(g, _):
        base = wid * EPW + g * CH
        pltpu.sync_copy(fdst_hbm.at[pl.ds(base, CH)], idx_v)
        for j in range(CH // 16):
            ii = idx_v[pl.ds(j * 16, 16)]
            w_v[pl.ds(j * 16, 16)] = plsc.load_gather(inv_v, [ii])
        pltpu.sync_copy(w_v, w_hbm.at[pl.ds(base, CH)])
        return 0
    lax.fori_loop(0, NCH_W, w_step, 0)


@functools.partial(
    pl.kernel,
    out_type=jax.ShapeDtypeStruct((NC, N, D), jnp.float32),
    mesh=_mesh,
    compiler_params=pltpu.CompilerParams(
        needs_layout_passes=False, use_tc_tiling_on_sc=False),
    scratch_types=[
        pltpu.VMEM_SHARED((N, D), jnp.float32),
        pltpu.VMEM((CH,), jnp.int32),
        pltpu.VMEM((CH,), jnp.int32),
        pltpu.VMEM((CH,), jnp.float32),
        pltpu.VMEM((CH, D), jnp.float32),
        pltpu.SemaphoreType.DMA,
    ],
)
def _agg_kernel(hflat_hbm, init_hbm, fsrc_hbm, dst_hbm, w_hbm, out_hbm,
                acc_sh, si_v, di_v, w_v, rows_v, sem):
    c = lax.axis_index("c")
    s = lax.axis_index("s")
    wid = s * NC + c
    r0 = s * RPT

    pltpu.sync_copy(init_hbm.at[c, pl.ds(r0, RPT)], acc_sh.at[pl.ds(r0, RPT)])
    plsc.subcore_barrier()

    def step(g, _):
        base = wid * EPW + g * CH
        pltpu.sync_copy(fsrc_hbm.at[pl.ds(base, CH)], si_v)
        pltpu.sync_copy(dst_hbm.at[pl.ds(base, CH)], di_v)
        pltpu.sync_copy(w_hbm.at[pl.ds(base, CH)], w_v)
        pltpu.async_copy(hflat_hbm.at[si_v], rows_v, sem).wait()

        def scale(j, _2):
            ws = plsc.load_gather(w_v, [lax.broadcast(j, (16,))])
            for cb in range(D // 16):
                rows_v[j, pl.ds(cb * 16, 16)] = rows_v[j, pl.ds(cb * 16, 16)] * ws
            return 0
        lax.fori_loop(0, CH, scale, 0)
        pltpu.sync_copy(rows_v, acc_sh.at[di_v], add=True)
        return 0
    lax.fori_loop(0, NCH_W, step, 0)
    plsc.subcore_barrier()

    pltpu.sync_copy(acc_sh.at[pl.ds(r0, RPT)], out_hbm.at[c, pl.ds(r0, RPT)])


def kernel(x, edge_index, edge_type, W_rel1, W_root1, b1, W_rel2, W_root2, b2):
    rows = E // D
    src2 = edge_index[0].reshape(rows, D)
    dst2 = edge_index[1].reshape(rows, D)
    et2 = edge_type.reshape(rows, D)
    fs, fd = _idx_prep(src2, dst2, et2)
    fs = fs.reshape(E)
    fd = fd.reshape(E)
    dst = edge_index[1]

    w = _weights_kernel(fd)

    zeros_nd = jnp.zeros((N, D), jnp.float32)

    w_all1 = jnp.concatenate([W_rel1, W_root1[None]], axis=0)
    h1 = _matmul_all(x, w_all1, b1)
    init1 = jnp.stack([h1[R], zeros_nd])
    acc1 = _agg_kernel(h1.reshape((R + 1) * N, D), init1, fs, dst, w)

    w_all2 = jnp.concatenate([W_rel2, W_root2[None]], axis=0)
    h2 = _matmul_all_relu(acc1, w_all2, b2)
    init2 = jnp.stack([h2[R], zeros_nd])
    acc2 = _agg_kernel(h2.reshape((R + 1) * N, D), init2, fs, dst, w)

    return _final_sum(acc2)


# speedup vs baseline: 25.5541x; 2.1880x over previous
import functools

import jax
import jax.numpy as jnp
from jax import lax
from jax.experimental import pallas as pl
from jax.experimental.pallas import tpu as pltpu
from jax.experimental.pallas import tpu_sc as plsc

N = 10000
E = 320000
D = 128
R = 8

NC = 2
NS = 16
NW = NC * NS
EPW = E // NW
CH = 80
NCH = EPW // CH
CHA = 40
NCHA = EPW // CHA
KB = 5
RN = R * N
RN_PAD = 81920
CNT_SLICE = RN_PAD // NS
RPT = N // NS

_mesh = plsc.VectorSubcoreMesh(core_axis_name="c", subcore_axis_name="s")
_sc_params = pltpu.CompilerParams(
    needs_layout_passes=False, use_tc_tiling_on_sc=False)


BN = 400
NB = N // BN


def _mm_body(x_ref, w_ref, b_ref, out_ref):
    r = pl.program_id(0)
    acc = jnp.dot(x_ref[...], w_ref[0], preferred_element_type=jnp.float32)
    bias = b_ref[0] * (r == R).astype(jnp.float32)
    out_ref[0] = acc + bias[None, :]


def _mm_relu_body(a_ref, w_ref, b_ref, out_ref):
    r = pl.program_id(0)
    xb = jnp.maximum(a_ref[0] + a_ref[1], 0.0)
    acc = jnp.dot(xb, w_ref[0], preferred_element_type=jnp.float32)
    bias = b_ref[0] * (r == R).astype(jnp.float32)
    out_ref[0] = acc + bias[None, :]


def _matmul_all(x, w_all, b):
    return pl.pallas_call(
        _mm_body,
        grid=(R + 1, NB),
        in_specs=[
            pl.BlockSpec((BN, D), lambda r, i: (i, 0)),
            pl.BlockSpec((1, D, D), lambda r, i: (r, 0, 0)),
            pl.BlockSpec((1, D), lambda r, i: (0, 0)),
        ],
        out_specs=pl.BlockSpec((1, BN, D), lambda r, i: (r, i, 0)),
        out_shape=jax.ShapeDtypeStruct((R + 1, N, D), jnp.float32),
    )(x, w_all, b[None])


def _matmul_all_relu(acc, w_all, b):
    return pl.pallas_call(
        _mm_relu_body,
        grid=(R + 1, NB),
        in_specs=[
            pl.BlockSpec((NC, BN, D), lambda r, i: (0, i, 0)),
            pl.BlockSpec((1, D, D), lambda r, i: (r, 0, 0)),
            pl.BlockSpec((1, D), lambda r, i: (0, 0)),
        ],
        out_specs=pl.BlockSpec((1, BN, D), lambda r, i: (r, i, 0)),
        out_shape=jax.ShapeDtypeStruct((R + 1, N, D), jnp.float32),
    )(acc, w_all, b[None])


def _idx_body(src_ref, dst_ref, et_ref, fs_ref, fd_ref):
    base = et_ref[...] * N
    fs_ref[...] = base + src_ref[...]
    fd_ref[...] = base + dst_ref[...]


def _idx_prep(src2, dst2, et2):
    rows = E // D
    return pl.pallas_call(
        _idx_body,
        out_shape=[jax.ShapeDtypeStruct((rows, D), jnp.int32)] * 2,
    )(src2, dst2, et2)


def _sum_body(a_ref, out_ref):
    out_ref[...] = a_ref[0] + a_ref[1]


def _final_sum(acc):
    return pl.pallas_call(
        _sum_body,
        grid=(NB,),
        in_specs=[pl.BlockSpec((NC, BN, D), lambda i: (0, i, 0))],
        out_specs=pl.BlockSpec((BN, D), lambda i: (i, 0)),
        out_shape=jax.ShapeDtypeStruct((N, D), jnp.float32),
    )(acc)


@functools.partial(
    pl.kernel,
    out_type=jax.ShapeDtypeStruct((NC, RN_PAD), jnp.float32),
    mesh=_mesh,
    compiler_params=_sc_params,
    scratch_types=[
        pltpu.VMEM_SHARED((RN_PAD,), jnp.float32),
        pltpu.VMEM((CNT_SLICE,), jnp.float32),
        pltpu.VMEM((CH,), jnp.float32),
        pltpu.VMEM((NCH, CH), jnp.int32),
        pltpu.SemaphoreType.DMA,
        pltpu.SemaphoreType.DMA,
    ],
)
def _count_kernel(fd3_hbm, cnt_hbm, cnt_sh, zbuf, ones_v, fdi_v, psem, ssem):
    c = lax.axis_index("c")
    s = lax.axis_index("s")
    wid = s * NC + c
    off = s * CNT_SLICE

    pltpu.async_copy(fd3_hbm.at[wid], fdi_v, psem)

    def z16(i, _):
        zbuf[pl.ds(i * 16, 16)] = jnp.zeros((16,), jnp.float32)
        return 0
    lax.fori_loop(0, CNT_SLICE // 16, z16, 0)
    pltpu.sync_copy(zbuf, cnt_sh.at[pl.ds(off, CNT_SLICE)])

    def o16(i, _):
        ones_v[pl.ds(i * 16, 16)] = jnp.ones((16,), jnp.float32)
        return 0
    lax.fori_loop(0, CH // 16, o16, 0)
    pltpu.make_async_copy(fd3_hbm.at[wid], fdi_v, psem).wait()
    plsc.subcore_barrier()

    GRP = 8

    def grp(i, _):
        for j in range(GRP):
            pltpu.async_copy(ones_v, cnt_sh.at[fdi_v.at[i * GRP + j]],
                             ssem, add=True)
        for j in range(GRP):
            pltpu.make_async_copy(ones_v, cnt_sh.at[fdi_v.at[0]], ssem).wait()
        return 0
    lax.fori_loop(0, NCH // GRP, grp, 0)
    for t in range((NCH // GRP) * GRP, NCH):
        pltpu.async_copy(ones_v, cnt_sh.at[fdi_v.at[t]], ssem, add=True)
    for t in range((NCH // GRP) * GRP, NCH):
        pltpu.make_async_copy(ones_v, cnt_sh.at[fdi_v.at[0]], ssem).wait()
    plsc.subcore_barrier()

    pltpu.sync_copy(cnt_sh.at[pl.ds(off, CNT_SLICE)],
                    cnt_hbm.at[c, pl.ds(off, CNT_SLICE)])


@functools.partial(
    pl.kernel,
    out_type=jax.ShapeDtypeStruct((NW, NCH, CH), jnp.float32),
    mesh=_mesh,
    compiler_params=_sc_params,
    scratch_types=[
        pltpu.VMEM_SHARED((RN_PAD,), jnp.float32),
        pltpu.VMEM((CNT_SLICE,), jnp.float32),
        pltpu.VMEM((CNT_SLICE,), jnp.float32),
        pltpu.VMEM((RN,), jnp.float32),
        pltpu.VMEM((NCH, CH), jnp.int32),
        pltpu.VMEM((NCH, CH), jnp.float32),
        pltpu.SemaphoreType.DMA,
    ],
)
def _wgt_kernel(cnt_hbm, fd3_hbm, w3_hbm, inv_sh, c0_v, c1_v, inv_v, fdi_v,
                wo_v, psem):
    c = lax.axis_index("c")
    s = lax.axis_index("s")
    wid = s * NC + c
    off = s * CNT_SLICE

    pltpu.async_copy(fd3_hbm.at[wid], fdi_v, psem)
    pltpu.sync_copy(cnt_hbm.at[0, pl.ds(off, CNT_SLICE)], c0_v)
    pltpu.sync_copy(cnt_hbm.at[1, pl.ds(off, CNT_SLICE)], c1_v)

    def inv16(i, _):
        v = c0_v[pl.ds(i * 16, 16)] + c1_v[pl.ds(i * 16, 16)]
        c0_v[pl.ds(i * 16, 16)] = 1.0 / jnp.maximum(v, 1.0)
        return 0
    lax.fori_loop(0, CNT_SLICE // 16, inv16, 0)
    pltpu.sync_copy(c0_v, inv_sh.at[pl.ds(off, CNT_SLICE)])
    plsc.subcore_barrier()

    pltpu.sync_copy(inv_sh.at[pl.ds(0, RN)], inv_v)
    pltpu.make_async_copy(fd3_hbm.at[wid], fdi_v, psem).wait()

    def wrow(g, _):
        for j in range(CH // 16):
            ii = fdi_v[g, pl.ds(j * 16, 16)]
            wo_v[g, pl.ds(j * 16, 16)] = plsc.load_gather(inv_v, [ii])
        return 0
    lax.fori_loop(0, NCH, wrow, 0)
    pltpu.sync_copy(wo_v, w3_hbm.at[wid])


@functools.partial(
    pl.kernel,
    out_type=jax.ShapeDtypeStruct((NC, N, D), jnp.float32),
    mesh=_mesh,
    compiler_params=_sc_params,
    scratch_types=(
        [pltpu.VMEM_SHARED((N, D), jnp.float32)]
        + [pltpu.VMEM((NCHA, CHA), jnp.int32)]
        + [pltpu.VMEM((CHA, D), jnp.float32)] * KB
        + [pltpu.VMEM((CHA,), jnp.int32)] * KB
        + [pltpu.VMEM((CHA,), jnp.float32)] * KB
        + [pltpu.SemaphoreType.DMA]
        + [pltpu.SemaphoreType.DMA] * KB
        + [pltpu.SemaphoreType.DMA] * KB
    ),
)
def _agg_kernel(hflat_hbm, init_hbm, fs3_hbm, dst3_hbm, w3_hbm, out_hbm,
                acc_sh, si_v, *slots):
    rows = slots[0:KB]
    dib = slots[KB:2 * KB]
    wb = slots[2 * KB:3 * KB]
    psem = slots[3 * KB]
    gsems = slots[3 * KB + 1:4 * KB + 1]
    ssems = slots[4 * KB + 1:5 * KB + 1]

    c = lax.axis_index("c")
    s = lax.axis_index("s")
    wid = s * NC + c
    row0 = s * RPT

    pltpu.async_copy(fs3_hbm.at[wid], si_v, psem)
    pltpu.async_copy(init_hbm.at[c, pl.ds(row0, RPT)],
                     acc_sh.at[pl.ds(row0, RPT)], psem)
    pltpu.make_async_copy(fs3_hbm.at[wid], si_v, psem).wait()
    pltpu.make_async_copy(init_hbm.at[c, pl.ds(row0, RPT)],
                          acc_sh.at[pl.ds(row0, RPT)], psem).wait()
    plsc.subcore_barrier()

    def fetch(t, b):
        pltpu.async_copy(dst3_hbm.at[wid, t], dib[b], gsems[b])
        pltpu.async_copy(w3_hbm.at[wid, t], wb[b], gsems[b])
        pltpu.async_copy(hflat_hbm.at[si_v.at[t]], rows[b], gsems[b])

    def fwait(t, b):
        pltpu.make_async_copy(dst3_hbm.at[wid, t], dib[b], gsems[b]).wait()
        pltpu.make_async_copy(w3_hbm.at[wid, t], wb[b], gsems[b]).wait()
        pltpu.make_async_copy(hflat_hbm.at[si_v.at[0]], rows[b],
                              gsems[b]).wait()

    def scat(b):
        pltpu.async_copy(rows[b], acc_sh.at[dib[b]], ssems[b], add=True)

    def swait(b):
        pltpu.make_async_copy(rows[b], acc_sh.at[dib[b]], ssems[b]).wait()

    def scale(b):
        def body(j, _):
            ws = plsc.load_gather(wb[b], [lax.broadcast(j, (16,))])
            r = rows[b]
            for cb in range(D // 16):
                r[j, pl.ds(cb * 16, 16)] = r[j, pl.ds(cb * 16, 16)] * ws
            return 0
        lax.fori_loop(0, CHA, body, 0)

    def step(t, b):
        fwait(t, b)
        scale(b)
        scat(b)
        bb = (b + KB - 1) % KB

        @pl.when(t >= 1)
        def _():
            swait(bb)

        @pl.when(t + KB - 1 < NCHA)
        def _():
            fetch(t + KB - 1, bb)

    for u in range(KB - 1):
        fetch(jnp.int32(u), u)

    def group(i, _):
        for k in range(KB):
            step(i * KB + k, k)
        return 0
    lax.fori_loop(0, NCHA // KB, group, 0)
    swait((NCHA - 1) % KB)

    plsc.subcore_barrier()
    pltpu.sync_copy(acc_sh.at[pl.ds(row0, RPT)],
                    out_hbm.at[c, pl.ds(row0, RPT)])


def kernel(x, edge_index, edge_type, W_rel1, W_root1, b1, W_rel2, W_root2, b2):
    rows = E // D
    src2 = edge_index[0].reshape(rows, D)
    dst2 = edge_index[1].reshape(rows, D)
    et2 = edge_type.reshape(rows, D)
    fs, fd = _idx_prep(src2, dst2, et2)
    fs3 = fs.reshape(NW, NCHA, CHA)
    fd3 = fd.reshape(NW, NCH, CH)
    dst3 = edge_index[1].reshape(NW, NCHA, CHA)

    cnt = _count_kernel(fd3)
    w3 = _wgt_kernel(cnt, fd3).reshape(NW, NCHA, CHA)

    zeros_nd = jnp.zeros((N, D), jnp.float32)

    w_all1 = jnp.concatenate([W_rel1, W_root1[None]], axis=0)
    h1 = _matmul_all(x, w_all1, b1)
    init1 = jnp.stack([h1[R], zeros_nd])
    acc1 = _agg_kernel(h1.reshape((R + 1) * N, D), init1, fs3, dst3, w3)

    w_all2 = jnp.concatenate([W_rel2, W_root2[None]], axis=0)
    h2 = _matmul_all_relu(acc1, w_all2, b2)
    init2 = jnp.stack([h2[R], zeros_nd])
    acc2 = _agg_kernel(h2.reshape((R + 1) * N, D), init2, fs3, dst3, w3)

    return _final_sum(acc2)
